# baseline (speedup 1.0000x reference)
import functools

import jax
import jax.numpy as jnp
from jax import lax
from jax.experimental import pallas as pl
from jax.experimental.pallas import tpu as pltpu
from jax.experimental.pallas import tpu_sc as plsc

_VB = 2048


def _argmax_partials_tc(x):
    R, V = x.shape
    nb = -(-V // _VB)
    nv = -(-nb // 2)
    tail = V - (nb - 1) * _VB

    def body(x_ref, pmax_ref, pidx_ref, m_ref, i_ref):
        h = pl.program_id(0)
        v = pl.program_id(1)

        @pl.when(v == 0)
        def _():
            m_ref[...] = jnp.full_like(m_ref, -jnp.inf)
            i_ref[...] = jnp.zeros_like(i_ref)

        bi = jnp.minimum(2 * v + h, nb - 1)

        def merge(xblk):
            m = jnp.max(xblk, axis=1, keepdims=True)
            itf = lax.broadcasted_iota(jnp.int32, xblk.shape, 1).astype(
                jnp.float32)
            cand = jnp.where(xblk == m, itf, jnp.float32(jnp.inf))
            li = jnp.min(cand, axis=1, keepdims=True).astype(jnp.int32)
            gi = li + bi * _VB
            pm = m_ref[...]
            pi = i_ref[...]
            better = (m > pm) | ((m == pm) & (gi < pi))
            m_ref[...] = jnp.where(better, m, pm)
            i_ref[...] = jnp.where(better, gi, pi)

        @pl.when(bi < nb - 1)
        def _():
            merge(x_ref[...])

        @pl.when(bi == nb - 1)
        def _():
            it = lax.broadcasted_iota(jnp.int32, x_ref.shape, 1)
            merge(jnp.where(it < tail, x_ref[...], -jnp.inf))

        @pl.when(v == nv - 1)
        def _():
            pmax_ref[...] = m_ref[...][None, :, :]
            pidx_ref[...] = i_ref[...][None, :, :]

    return pl.pallas_call(
        body,
        grid=(2, nv),
        in_specs=[pl.BlockSpec(
            (R, _VB),
            lambda h, v: (jnp.int32(0), jnp.minimum(2 * v + h, nb - 1)))],
        out_specs=[
            pl.BlockSpec((1, R, 1), lambda h, v: (h, jnp.int32(0), jnp.int32(0))),
            pl.BlockSpec((1, R, 1), lambda h, v: (h, jnp.int32(0), jnp.int32(0)))],
        out_shape=[jax.ShapeDtypeStruct((2, R, 1), jnp.float32),
                   jax.ShapeDtypeStruct((2, R, 1), jnp.int32)],
        scratch_shapes=[pltpu.VMEM((R, 1), jnp.float32),
                        pltpu.VMEM((R, 1), jnp.int32)],
        compiler_params=pltpu.CompilerParams(
            dimension_semantics=("parallel", "arbitrary")),
    )(x)


def _rejection_sc(pmax, pidx, draft, nd, ndeff, bonus):
    R = draft.shape[0]
    B = nd.shape[0]
    S = R // B
    L = 16
    mesh = plsc.VectorSubcoreMesh(core_axis_name="c", subcore_axis_name="s")

    @functools.partial(
        pl.kernel, mesh=mesh,
        compiler_params=pltpu.CompilerParams(needs_layout_passes=False),
        out_type=[jax.ShapeDtypeStruct((B, S + 1), jnp.int32),
                  jax.ShapeDtypeStruct((B,), jnp.int32),
                  jax.ShapeDtypeStruct((B,), jnp.int32)],
        scratch_types=[pltpu.VMEM((2 * R,), jnp.float32),
                       pltpu.VMEM((2 * R,), jnp.int32),
                       pltpu.VMEM((R,), jnp.int32),
                       pltpu.VMEM((R,), jnp.int32),
                       pltpu.VMEM((B,), jnp.int32),
                       pltpu.VMEM((B,), jnp.int32),
                       pltpu.VMEM((B,), jnp.int32),
                       pltpu.VMEM((B, S + 1), jnp.int32),
                       pltpu.VMEM((B,), jnp.int32),
                       pltpu.VMEM((B,), jnp.int32)],
    )
    def k(pmax_hbm, pidx_hbm, draft_hbm, nd_hbm, ndeff_hbm, bonus_hbm,
          out_hbm, nrej_hbm, last_hbm,
          pmax_v, pidx_v, draft_v, amax_v, nd_v, ndeff_v, bonus_v,
          out_v, nrej_v, last_v):
        cid = lax.axis_index("c")
        sid = lax.axis_index("s")

        @pl.when((cid == 0) & (sid == 0))
        def _():
            pltpu.sync_copy(pmax_hbm, pmax_v)
            pltpu.sync_copy(pidx_hbm, pidx_v)
            pltpu.sync_copy(draft_hbm, draft_v)
            pltpu.sync_copy(nd_hbm, nd_v)
            pltpu.sync_copy(ndeff_hbm, ndeff_v)
            pltpu.sync_copy(bonus_hbm, bonus_v)
            i16 = jnp.arange(L, dtype=jnp.int32)
            z16 = jnp.zeros((L,), jnp.int32)

            for i in range(R // L):
                rows = i16 + (L * i)
                m0 = plsc.load_gather(pmax_v, [rows])
                m1 = plsc.load_gather(pmax_v, [rows + R])
                i0 = plsc.load_gather(pidx_v, [rows])
                i1 = plsc.load_gather(pidx_v, [rows + R])
                take1 = (m1 > m0) | ((m1 == m0) & (i1 < i0))
                amax_v[pl.ds(L * i, L)] = jnp.where(take1, i1, i0)

            carry = jnp.int32(0)
            for i in range(B // L):
                sl = pl.ds(L * i, L)
                ndc = nd_v[sl]
                ndeffc = ndeff_v[sl]
                bonusc = bonus_v[sl]
                inc = plsc.cumsum(ndc)
                cu = inc - ndc + carry
                carry = carry + jnp.max(inc)

                tvals = []
                na = jnp.full((L,), S, jnp.int32)
                for s in range(S):
                    idxt = jnp.clip(cu + s, 0, R - 1)
                    tg = plsc.load_gather(amax_v, [idxt])
                    dr = plsc.load_gather(draft_v, [idxt])
                    tvals.append(tg)
                    match = (tg == dr) & (jnp.full((L,), s, jnp.int32) < ndeffc)
                    na = jnp.minimum(na, jnp.where(
                        match, jnp.full((L,), S, jnp.int32),
                        jnp.full((L,), s, jnp.int32)))

                all_acc = na == ndc
                one = jnp.full((L,), 1, jnp.int32)
                zero = jnp.zeros((L,), jnp.int32)
                nst = na + jnp.where(all_acc, zero, one)
                nrej_v[sl] = ndc - na

                lastsel = jnp.clip(nst - 1, 0, S - 1)
                lastt = zero
                for s in range(S):
                    lastt = jnp.where(
                        lastsel == jnp.full((L,), s, jnp.int32),
                        tvals[s], lastt)
                last_v[sl] = jnp.where(all_acc, bonusc, lastt)

                bvec = i16 + (L * i)
                neg1 = jnp.full((L,), -1, jnp.int32)
                for j in range(S + 1):
                    jv = jnp.full((L,), j, jnp.int32)
                    if j < S:
                        row = jnp.where(
                            jv < nst, tvals[j],
                            jnp.where(all_acc & (ndc == jv), bonusc, neg1))
                    else:
                        row = jnp.where(all_acc & (ndc == jv), bonusc, neg1)
                    plsc.store_scatter(out_v, [bvec, jv], row)

            pltpu.sync_copy(out_v, out_hbm)
            pltpu.sync_copy(nrej_v, nrej_hbm)
            pltpu.sync_copy(last_v, last_hbm)

    return k(pmax, pidx, draft, nd, ndeff, bonus)


def kernel(target_logits, draft_token_ids, bonus_token_ids, num_draft_tokens,
           max_spec_num):
    draft = draft_token_ids.astype(jnp.int32)
    bonus = bonus_token_ids.astype(jnp.int32)
    nd = num_draft_tokens.astype(jnp.int32)
    ndeff = jnp.minimum(nd, jnp.asarray(max_spec_num).astype(jnp.int32))

    pmax, pidx = _argmax_partials_tc(target_logits.astype(jnp.float32))
    out32, nrej32, last32 = _rejection_sc(
        pmax.reshape(-1), pidx.reshape(-1), draft, nd, ndeff, bonus)

    out = out32.astype(bonus_token_ids.dtype)
    num_rejected = nrej32.astype(num_draft_tokens.dtype)
    last_token_ids = last32.astype(num_draft_tokens.dtype)
    return (out, num_rejected, last_token_ids)
